# baseline (device time: 35129 ns/iter reference)
import jax
import jax.numpy as jnp
from jax import lax
from jax.experimental import pallas as pl
from jax.experimental.pallas import tpu as pltpu

EPS = 1e-6
CHUNK_ROWS = (32, 96, 64, 64)
CHUNK_OFF = (0, 32, 128, 192)
NC = len(CHUNK_ROWS)


def kernel(partial, gamma):
    g = gamma.reshape(1, -1)
    _, m2, d = partial.shape
    m = m2 // 2
    qr = m // 4

    def body(
        p_ref, g_ref, out_ref, recv_x, local_p,
        local_sem, x_send, x_recv,
        ydir_send, ydir_recv, zdir_send, zdir_recv,
        yfwd_send, yfwd_recv, zfwd_send, zfwd_recv,
    ):
        my_x = lax.axis_index("x")
        my_y = lax.axis_index("y")
        my_z = lax.axis_index("z")
        peer = (1 - my_x, my_y, my_z)
        nbr_y = (my_x, my_y ^ 1, my_z)
        nbr_z = (my_x, my_y, my_z ^ 1)
        a = my_y % 2
        b = my_z % 2
        q = 2 * a + b
        qy = q ^ 2
        qz = q ^ 1
        qd = q ^ 3

        def row(qi, c):
            return qi * qr + CHUNK_OFF[c]

        local_copies = []
        for c in range(NC):
            cp = pltpu.make_async_copy(
                p_ref.at[0, pl.ds(my_x * m + row(q, c), CHUNK_ROWS[c]), :],
                local_p.at[pl.ds(CHUNK_OFF[c], CHUNK_ROWS[c]), :],
                local_sem.at[c],
            )
            cp.start()
            local_copies.append(cp)

        barrier = pltpu.get_barrier_semaphore()
        for dev in (peer, nbr_y, nbr_z):
            pl.semaphore_signal(
                barrier, inc=1, device_id=dev,
                device_id_type=pl.DeviceIdType.MESH,
            )
        pl.semaphore_wait(barrier, 3)

        x_rdmas = []
        for c in range(NC):
            rdma = pltpu.make_async_remote_copy(
                src_ref=p_ref.at[
                    0, pl.ds((1 - my_x) * m + row(q, c), CHUNK_ROWS[c]), :
                ],
                dst_ref=recv_x.at[pl.ds(CHUNK_OFF[c], CHUNK_ROWS[c]), :],
                send_sem=x_send.at[c],
                recv_sem=x_recv.at[c],
                device_id=peer,
                device_id_type=pl.DeviceIdType.MESH,
            )
            rdma.start()
            x_rdmas.append(rdma)

        def block_rdma(qi, c, dev, send_sem, recv_sem):
            return pltpu.make_async_remote_copy(
                src_ref=out_ref.at[pl.ds(row(qi, c), CHUNK_ROWS[c]), :],
                dst_ref=out_ref.at[pl.ds(row(qi, c), CHUNK_ROWS[c]), :],
                send_sem=send_sem,
                recv_sem=recv_sem,
                device_id=dev,
                device_id_type=pl.DeviceIdType.MESH,
            )

        dir_rdmas = []
        for c in range(NC):
            local_copies[c].wait()
            x_rdmas[c].wait_recv()
            sl = pl.ds(CHUNK_OFF[c], CHUNK_ROWS[c])
            y = local_p[sl, :] + recv_x[sl, :]
            ms = jnp.mean(y * y, axis=-1, keepdims=True)
            out_ref[pl.ds(row(q, c), CHUNK_ROWS[c]), :] = (
                y * lax.rsqrt(ms + EPS) * g_ref[0, :]
            )
            ry = block_rdma(q, c, nbr_y, ydir_send.at[c], ydir_recv.at[c])
            ry.start()
            rz = block_rdma(q, c, nbr_z, zdir_send.at[c], zdir_recv.at[c])
            rz.start()
            dir_rdmas.append((ry, rz))

        zdir_in = [
            block_rdma(qz, c, nbr_z, zdir_send.at[c], zdir_recv.at[c])
            for c in range(NC)
        ]
        ydir_in = [
            block_rdma(qy, c, nbr_y, ydir_send.at[c], ydir_recv.at[c])
            for c in range(NC)
        ]
        h0 = range(NC // 2)
        h1 = range(NC // 2, NC)
        fwd_rdmas = []
        for c in h0:
            zdir_in[c].wait_recv()
            r = block_rdma(qz, c, nbr_y, yfwd_send.at[c], yfwd_recv.at[c])
            r.start()
            fwd_rdmas.append(r)
        for c in h1:
            ydir_in[c].wait_recv()
            r = block_rdma(qy, c, nbr_z, zfwd_send.at[c - NC // 2],
                           zfwd_recv.at[c - NC // 2])
            r.start()
            fwd_rdmas.append(r)

        for c in h0:
            ydir_in[c].wait_recv()
        for c in h1:
            zdir_in[c].wait_recv()
        for c in h0:
            block_rdma(qd, c, nbr_y, yfwd_send.at[c], yfwd_recv.at[c]).wait_recv()
        for c in h1:
            block_rdma(qd, c, nbr_z, zfwd_send.at[c - NC // 2],
                       zfwd_recv.at[c - NC // 2]).wait_recv()
        for c in range(NC):
            x_rdmas[c].wait_send()
            dir_rdmas[c][0].wait_send()
            dir_rdmas[c][1].wait_send()
        for r in fwd_rdmas:
            r.wait_send()

    n_sem = pltpu.SemaphoreType.DMA
    return pl.pallas_call(
        body,
        out_shape=jax.ShapeDtypeStruct((m, d), jnp.float32),
        in_specs=[
            pl.BlockSpec(memory_space=pl.ANY),
            pl.BlockSpec(memory_space=pltpu.VMEM),
        ],
        out_specs=pl.BlockSpec(memory_space=pltpu.VMEM),
        scratch_shapes=[
            pltpu.VMEM((qr, d), jnp.float32),
            pltpu.VMEM((qr, d), jnp.float32),
            n_sem((NC,)), n_sem((NC,)), n_sem((NC,)),
            n_sem((NC,)), n_sem((NC,)), n_sem((NC,)), n_sem((NC,)),
            n_sem((NC // 2,)), n_sem((NC // 2,)),
            n_sem((NC // 2,)), n_sem((NC // 2,)),
        ],
        compiler_params=pltpu.CompilerParams(collective_id=0),
    )(partial, g)


# device time: 30210 ns/iter; 1.1628x vs baseline; 1.1628x over previous
import jax
import jax.numpy as jnp
from jax import lax
from jax.experimental import pallas as pl
from jax.experimental.pallas import tpu as pltpu

EPS = 1e-6
CHUNK_ROWS = (64, 64, 64, 64)
CHUNK_OFF = (0, 64, 128, 192)
NC = len(CHUNK_ROWS)


def kernel(partial, gamma):
    g = gamma.reshape(1, -1)
    _, m2, d = partial.shape
    m = m2 // 2
    qr = m // 4

    def body(
        p_ref, g_ref, out_ref, recv_x, local_p, stage,
        local_sem, stage_sem, x_send, x_recv,
        ydir_send, ydir_recv, zdir_send, zdir_recv,
        yfwd_send, yfwd_recv, zfwd_send, zfwd_recv,
    ):
        my_x = lax.axis_index("x")
        my_y = lax.axis_index("y")
        my_z = lax.axis_index("z")
        peer = (1 - my_x, my_y, my_z)
        nbr_y = (my_x, my_y ^ 1, my_z)
        nbr_z = (my_x, my_y, my_z ^ 1)
        a = my_y % 2
        b = my_z % 2
        q = 2 * a + b
        qy = q ^ 2
        qz = q ^ 1
        qd = q ^ 3

        def row(qi, c):
            return qi * qr + CHUNK_OFF[c]

        local_copies = []
        for c in range(NC):
            cp = pltpu.make_async_copy(
                p_ref.at[0, pl.ds(my_x * m + row(q, c), CHUNK_ROWS[c]), :],
                local_p.at[pl.ds(CHUNK_OFF[c], CHUNK_ROWS[c]), :],
                local_sem.at[c],
            )
            cp.start()
            local_copies.append(cp)

        barrier = pltpu.get_barrier_semaphore()
        for dev in (peer, nbr_y, nbr_z):
            pl.semaphore_signal(
                barrier, inc=1, device_id=dev,
                device_id_type=pl.DeviceIdType.MESH,
            )
        pl.semaphore_wait(barrier, 3)

        x_rdmas = []
        for c in range(NC):
            rdma = pltpu.make_async_remote_copy(
                src_ref=p_ref.at[
                    0, pl.ds((1 - my_x) * m + row(q, c), CHUNK_ROWS[c]), :
                ],
                dst_ref=recv_x.at[pl.ds(CHUNK_OFF[c], CHUNK_ROWS[c]), :],
                send_sem=x_send.at[c],
                recv_sem=x_recv.at[c],
                device_id=peer,
                device_id_type=pl.DeviceIdType.MESH,
            )
            rdma.start()
            x_rdmas.append(rdma)

        def gather_rdma(src_ref, qi, c, dev, send_sem, recv_sem):
            return pltpu.make_async_remote_copy(
                src_ref=src_ref,
                dst_ref=out_ref.at[pl.ds(row(qi, c), CHUNK_ROWS[c]), :],
                send_sem=send_sem,
                recv_sem=recv_sem,
                device_id=dev,
                device_id_type=pl.DeviceIdType.MESH,
            )

        def out_src(qi, c):
            return out_ref.at[pl.ds(row(qi, c), CHUNK_ROWS[c]), :]

        dir_rdmas = []
        stage_copies = []
        for c in range(NC):
            sl = pl.ds(CHUNK_OFF[c], CHUNK_ROWS[c])
            local_copies[c].wait()
            x_rdmas[c].wait_recv()
            y = local_p[sl, :] + recv_x[sl, :]
            ms = jnp.mean(y * y, axis=-1, keepdims=True)
            stage[sl, :] = y * lax.rsqrt(ms + EPS) * g_ref[0, :]
            cp = pltpu.make_async_copy(
                stage.at[sl, :],
                out_ref.at[pl.ds(row(q, c), CHUNK_ROWS[c]), :],
                stage_sem.at[c],
            )
            cp.start()
            stage_copies.append(cp)
            ry = gather_rdma(stage.at[sl, :], q, c, nbr_y,
                             ydir_send.at[c], ydir_recv.at[c])
            ry.start()
            rz = gather_rdma(stage.at[sl, :], q, c, nbr_z,
                             zdir_send.at[c], zdir_recv.at[c])
            rz.start()
            dir_rdmas.append((ry, rz))

        zdir_in = [
            gather_rdma(out_src(qz, c), qz, c, nbr_z,
                        zdir_send.at[c], zdir_recv.at[c])
            for c in range(NC)
        ]
        ydir_in = [
            gather_rdma(out_src(qy, c), qy, c, nbr_y,
                        ydir_send.at[c], ydir_recv.at[c])
            for c in range(NC)
        ]
        h0 = range(NC // 2)
        h1 = range(NC // 2, NC)
        fwd_rdmas = []
        for c in h0:
            zdir_in[c].wait_recv()
            r = gather_rdma(out_src(qz, c), qz, c, nbr_y,
                            yfwd_send.at[c], yfwd_recv.at[c])
            r.start()
            fwd_rdmas.append(r)
        for c in h1:
            ydir_in[c].wait_recv()
            r = gather_rdma(out_src(qy, c), qy, c, nbr_z,
                            zfwd_send.at[c - NC // 2],
                            zfwd_recv.at[c - NC // 2])
            r.start()
            fwd_rdmas.append(r)

        for c in h0:
            ydir_in[c].wait_recv()
        for c in h1:
            zdir_in[c].wait_recv()
        for c in h0:
            gather_rdma(out_src(qd, c), qd, c, nbr_y,
                        yfwd_send.at[c], yfwd_recv.at[c]).wait_recv()
        for c in h1:
            gather_rdma(out_src(qd, c), qd, c, nbr_z,
                        zfwd_send.at[c - NC // 2],
                        zfwd_recv.at[c - NC // 2]).wait_recv()
        for c in range(NC):
            stage_copies[c].wait()
            x_rdmas[c].wait_send()
            dir_rdmas[c][0].wait_send()
            dir_rdmas[c][1].wait_send()
        for r in fwd_rdmas:
            r.wait_send()

    n_sem = pltpu.SemaphoreType.DMA
    p_hbm = pltpu.with_memory_space_constraint(partial, pltpu.MemorySpace.HBM)
    g_hbm = pltpu.with_memory_space_constraint(g, pltpu.MemorySpace.HBM)
    return pl.pallas_call(
        body,
        out_shape=jax.ShapeDtypeStruct((m, d), jnp.float32),
        in_specs=[
            pl.BlockSpec(memory_space=pl.ANY),
            pl.BlockSpec(memory_space=pltpu.VMEM),
        ],
        out_specs=pl.BlockSpec(memory_space=pl.ANY),
        scratch_shapes=[
            pltpu.VMEM((qr, d), jnp.float32),
            pltpu.VMEM((qr, d), jnp.float32),
            pltpu.VMEM((qr, d), jnp.float32),
            n_sem((NC,)), n_sem((NC,)), n_sem((NC,)), n_sem((NC,)),
            n_sem((NC,)), n_sem((NC,)), n_sem((NC,)), n_sem((NC,)),
            n_sem((NC // 2,)), n_sem((NC // 2,)),
            n_sem((NC // 2,)), n_sem((NC // 2,)),
        ],
        compiler_params=pltpu.CompilerParams(collective_id=0),
    )(p_hbm, g_hbm)


# device time: 30025 ns/iter; 1.1700x vs baseline; 1.0062x over previous
import jax
import jax.numpy as jnp
from jax import lax
from jax.experimental import pallas as pl
from jax.experimental.pallas import tpu as pltpu

EPS = 1e-6
CHUNK_ROWS = (64, 64, 64, 64)
CHUNK_OFF = (0, 64, 128, 192)
NC = len(CHUNK_ROWS)


def kernel(partial, gamma):
    g = gamma.reshape(1, -1)
    _, m2, d = partial.shape
    m = m2 // 2
    qr = m // 4

    def body(
        p_ref, g_ref, out_ref, recv_x, local_p, stage, send0,
        local_sem, stage_sem, send0_sem, x_send, x_recv,
        ydir_send, ydir_recv, zdir_send, zdir_recv,
        yfwd_send, yfwd_recv, zfwd_send, zfwd_recv,
    ):
        my_x = lax.axis_index("x")
        my_y = lax.axis_index("y")
        my_z = lax.axis_index("z")
        peer = (1 - my_x, my_y, my_z)
        nbr_y = (my_x, my_y ^ 1, my_z)
        nbr_z = (my_x, my_y, my_z ^ 1)
        a = my_y % 2
        b = my_z % 2
        q = 2 * a + b
        qy = q ^ 2
        qz = q ^ 1
        qd = q ^ 3

        def row(qi, c):
            return qi * qr + CHUNK_OFF[c]

        send0_cp = pltpu.make_async_copy(
            p_ref.at[0, pl.ds((1 - my_x) * m + row(q, 0), CHUNK_ROWS[0]), :],
            send0,
            send0_sem,
        )
        send0_cp.start()

        local_copies = []
        for c in range(NC):
            cp = pltpu.make_async_copy(
                p_ref.at[0, pl.ds(my_x * m + row(q, c), CHUNK_ROWS[c]), :],
                local_p.at[pl.ds(CHUNK_OFF[c], CHUNK_ROWS[c]), :],
                local_sem.at[c],
            )
            cp.start()
            local_copies.append(cp)

        barrier = pltpu.get_barrier_semaphore()
        for dev in (peer, nbr_y, nbr_z):
            pl.semaphore_signal(
                barrier, inc=1, device_id=dev,
                device_id_type=pl.DeviceIdType.MESH,
            )
        pl.semaphore_wait(barrier, 3)

        send0_cp.wait()
        x_rdmas = []
        for c in range(NC):
            src = (
                send0.at[:, :]
                if c == 0
                else p_ref.at[
                    0, pl.ds((1 - my_x) * m + row(q, c), CHUNK_ROWS[c]), :
                ]
            )
            rdma = pltpu.make_async_remote_copy(
                src_ref=src,
                dst_ref=recv_x.at[pl.ds(CHUNK_OFF[c], CHUNK_ROWS[c]), :],
                send_sem=x_send.at[c],
                recv_sem=x_recv.at[c],
                device_id=peer,
                device_id_type=pl.DeviceIdType.MESH,
            )
            rdma.start()
            x_rdmas.append(rdma)

        def gather_rdma(src_ref, qi, c, dev, send_sem, recv_sem):
            return pltpu.make_async_remote_copy(
                src_ref=src_ref,
                dst_ref=out_ref.at[pl.ds(row(qi, c), CHUNK_ROWS[c]), :],
                send_sem=send_sem,
                recv_sem=recv_sem,
                device_id=dev,
                device_id_type=pl.DeviceIdType.MESH,
            )

        def out_src(qi, c):
            return out_ref.at[pl.ds(row(qi, c), CHUNK_ROWS[c]), :]

        dir_rdmas = []
        stage_copies = []
        for c in range(NC):
            sl = pl.ds(CHUNK_OFF[c], CHUNK_ROWS[c])
            local_copies[c].wait()
            x_rdmas[c].wait_recv()
            y = local_p[sl, :] + recv_x[sl, :]
            ms = jnp.mean(y * y, axis=-1, keepdims=True)
            stage[sl, :] = y * lax.rsqrt(ms + EPS) * g_ref[0, :]
            cp = pltpu.make_async_copy(
                stage.at[sl, :],
                out_ref.at[pl.ds(row(q, c), CHUNK_ROWS[c]), :],
                stage_sem.at[c],
            )
            cp.start()
            stage_copies.append(cp)
            ry = gather_rdma(stage.at[sl, :], q, c, nbr_y,
                             ydir_send.at[c], ydir_recv.at[c])
            ry.start()
            rz = gather_rdma(stage.at[sl, :], q, c, nbr_z,
                             zdir_send.at[c], zdir_recv.at[c])
            rz.start()
            dir_rdmas.append((ry, rz))

        zdir_in = [
            gather_rdma(out_src(qz, c), qz, c, nbr_z,
                        zdir_send.at[c], zdir_recv.at[c])
            for c in range(NC)
        ]
        ydir_in = [
            gather_rdma(out_src(qy, c), qy, c, nbr_y,
                        ydir_send.at[c], ydir_recv.at[c])
            for c in range(NC)
        ]
        h0 = range(NC // 2)
        h1 = range(NC // 2, NC)
        fwd_rdmas = []
        for c in h0:
            zdir_in[c].wait_recv()
            r = gather_rdma(out_src(qz, c), qz, c, nbr_y,
                            yfwd_send.at[c], yfwd_recv.at[c])
            r.start()
            fwd_rdmas.append(r)
        for c in h1:
            ydir_in[c].wait_recv()
            r = gather_rdma(out_src(qy, c), qy, c, nbr_z,
                            zfwd_send.at[c - NC // 2],
                            zfwd_recv.at[c - NC // 2])
            r.start()
            fwd_rdmas.append(r)

        for c in h0:
            ydir_in[c].wait_recv()
        for c in h1:
            zdir_in[c].wait_recv()
        for c in h0:
            gather_rdma(out_src(qd, c), qd, c, nbr_y,
                        yfwd_send.at[c], yfwd_recv.at[c]).wait_recv()
        for c in h1:
            gather_rdma(out_src(qd, c), qd, c, nbr_z,
                        zfwd_send.at[c - NC // 2],
                        zfwd_recv.at[c - NC // 2]).wait_recv()
        for c in range(NC):
            stage_copies[c].wait()
            x_rdmas[c].wait_send()
            dir_rdmas[c][0].wait_send()
            dir_rdmas[c][1].wait_send()
        for r in fwd_rdmas:
            r.wait_send()

    n_sem = pltpu.SemaphoreType.DMA
    p_hbm = pltpu.with_memory_space_constraint(partial, pltpu.MemorySpace.HBM)
    g_hbm = pltpu.with_memory_space_constraint(g, pltpu.MemorySpace.HBM)
    return pl.pallas_call(
        body,
        out_shape=jax.ShapeDtypeStruct((m, d), jnp.float32),
        in_specs=[
            pl.BlockSpec(memory_space=pl.ANY),
            pl.BlockSpec(memory_space=pltpu.VMEM),
        ],
        out_specs=pl.BlockSpec(memory_space=pl.ANY),
        scratch_shapes=[
            pltpu.VMEM((qr, d), jnp.float32),
            pltpu.VMEM((qr, d), jnp.float32),
            pltpu.VMEM((qr, d), jnp.float32),
            pltpu.VMEM((CHUNK_ROWS[0], d), jnp.float32),
            n_sem((NC,)), n_sem((NC,)), n_sem, n_sem((NC,)), n_sem((NC,)),
            n_sem((NC,)), n_sem((NC,)), n_sem((NC,)), n_sem((NC,)),
            n_sem((NC // 2,)), n_sem((NC // 2,)),
            n_sem((NC // 2,)), n_sem((NC // 2,)),
        ],
        compiler_params=pltpu.CompilerParams(collective_id=0),
    )(p_hbm, g_hbm)


# device time: 29936 ns/iter; 1.1735x vs baseline; 1.0030x over previous
import jax
import jax.numpy as jnp
from jax import lax
from jax.experimental import pallas as pl
from jax.experimental.pallas import tpu as pltpu

EPS = 1e-6
CHUNK_ROWS = (64, 64, 64, 64)
CHUNK_OFF = (0, 64, 128, 192)
NC = len(CHUNK_ROWS)


def kernel(partial, gamma):
    g = gamma.reshape(1, -1)
    _, m2, d = partial.shape
    m = m2 // 2
    qr = m // 4

    def body(
        p_ref, g_ref, out_ref, recv_x, local_p, send0,
        local_sem, send0_sem, x_send, x_recv,
        ydir_send, ydir_recv, zdir_send, zdir_recv,
        yfwd_send, yfwd_recv, zfwd_send, zfwd_recv,
    ):
        my_x = lax.axis_index("x")
        my_y = lax.axis_index("y")
        my_z = lax.axis_index("z")
        peer = (1 - my_x, my_y, my_z)
        nbr_y = (my_x, my_y ^ 1, my_z)
        nbr_z = (my_x, my_y, my_z ^ 1)
        a = my_y % 2
        b = my_z % 2
        q = 2 * a + b
        qy = q ^ 2
        qz = q ^ 1
        qd = q ^ 3

        def row(qi, c):
            return qi * qr + CHUNK_OFF[c]

        send0_cp = pltpu.make_async_copy(
            p_ref.at[0, pl.ds((1 - my_x) * m + row(q, 0), CHUNK_ROWS[0]), :],
            send0,
            send0_sem,
        )
        send0_cp.start()

        local_copies = []
        for c in range(NC):
            cp = pltpu.make_async_copy(
                p_ref.at[0, pl.ds(my_x * m + row(q, c), CHUNK_ROWS[c]), :],
                local_p.at[pl.ds(CHUNK_OFF[c], CHUNK_ROWS[c]), :],
                local_sem.at[c],
            )
            cp.start()
            local_copies.append(cp)

        barrier = pltpu.get_barrier_semaphore()
        for dev in (peer, nbr_y, nbr_z):
            pl.semaphore_signal(
                barrier, inc=1, device_id=dev,
                device_id_type=pl.DeviceIdType.MESH,
            )
        pl.semaphore_wait(barrier, 3)

        send0_cp.wait()
        x_rdmas = []
        for c in range(NC):
            src = (
                send0.at[:, :]
                if c == 0
                else p_ref.at[
                    0, pl.ds((1 - my_x) * m + row(q, c), CHUNK_ROWS[c]), :
                ]
            )
            rdma = pltpu.make_async_remote_copy(
                src_ref=src,
                dst_ref=recv_x.at[pl.ds(CHUNK_OFF[c], CHUNK_ROWS[c]), :],
                send_sem=x_send.at[c],
                recv_sem=x_recv.at[c],
                device_id=peer,
                device_id_type=pl.DeviceIdType.MESH,
            )
            rdma.start()
            x_rdmas.append(rdma)

        def gather_rdma(src_ref, qi, c, dev, send_sem, recv_sem):
            return pltpu.make_async_remote_copy(
                src_ref=src_ref,
                dst_ref=out_ref.at[pl.ds(row(qi, c), CHUNK_ROWS[c]), :],
                send_sem=send_sem,
                recv_sem=recv_sem,
                device_id=dev,
                device_id_type=pl.DeviceIdType.MESH,
            )

        def out_src(qi, c):
            return out_ref.at[pl.ds(row(qi, c), CHUNK_ROWS[c]), :]

        dir_rdmas = []
        for c in range(NC):
            sl = pl.ds(CHUNK_OFF[c], CHUNK_ROWS[c])
            local_copies[c].wait()
            x_rdmas[c].wait_recv()
            y = local_p[sl, :] + recv_x[sl, :]
            ms = jnp.mean(y * y, axis=-1, keepdims=True)
            out_ref[pl.ds(row(q, c), CHUNK_ROWS[c]), :] = (
                y * lax.rsqrt(ms + EPS) * g_ref[0, :]
            )
            ry = gather_rdma(out_src(q, c), q, c, nbr_y,
                             ydir_send.at[c], ydir_recv.at[c])
            ry.start()
            rz = gather_rdma(out_src(q, c), q, c, nbr_z,
                             zdir_send.at[c], zdir_recv.at[c])
            rz.start()
            dir_rdmas.append((ry, rz))

        zdir_in = [
            gather_rdma(out_src(qz, c), qz, c, nbr_z,
                        zdir_send.at[c], zdir_recv.at[c])
            for c in range(NC)
        ]
        ydir_in = [
            gather_rdma(out_src(qy, c), qy, c, nbr_y,
                        ydir_send.at[c], ydir_recv.at[c])
            for c in range(NC)
        ]
        h0 = range(NC // 2)
        h1 = range(NC // 2, NC)
        fwd_rdmas = []
        for c in h0:
            zdir_in[c].wait_recv()
            r = gather_rdma(out_src(qz, c), qz, c, nbr_y,
                            yfwd_send.at[c], yfwd_recv.at[c])
            r.start()
            fwd_rdmas.append(r)
        for c in h1:
            ydir_in[c].wait_recv()
            r = gather_rdma(out_src(qy, c), qy, c, nbr_z,
                            zfwd_send.at[c - NC // 2],
                            zfwd_recv.at[c - NC // 2])
            r.start()
            fwd_rdmas.append(r)

        for c in h0:
            ydir_in[c].wait_recv()
        for c in h1:
            zdir_in[c].wait_recv()
        for c in h0:
            gather_rdma(out_src(qd, c), qd, c, nbr_y,
                        yfwd_send.at[c], yfwd_recv.at[c]).wait_recv()
        for c in h1:
            gather_rdma(out_src(qd, c), qd, c, nbr_z,
                        zfwd_send.at[c - NC // 2],
                        zfwd_recv.at[c - NC // 2]).wait_recv()
        for c in range(NC):
            x_rdmas[c].wait_send()
            dir_rdmas[c][0].wait_send()
            dir_rdmas[c][1].wait_send()
        for r in fwd_rdmas:
            r.wait_send()

    n_sem = pltpu.SemaphoreType.DMA
    p_hbm = pltpu.with_memory_space_constraint(partial, pltpu.MemorySpace.HBM)
    g_hbm = pltpu.with_memory_space_constraint(g, pltpu.MemorySpace.HBM)
    return pl.pallas_call(
        body,
        out_shape=jax.ShapeDtypeStruct((m, d), jnp.float32),
        in_specs=[
            pl.BlockSpec(memory_space=pl.ANY),
            pl.BlockSpec(memory_space=pltpu.VMEM),
        ],
        out_specs=pl.BlockSpec(memory_space=pltpu.VMEM),
        scratch_shapes=[
            pltpu.VMEM((qr, d), jnp.float32),
            pltpu.VMEM((qr, d), jnp.float32),
            pltpu.VMEM((CHUNK_ROWS[0], d), jnp.float32),
            n_sem((NC,)), n_sem, n_sem((NC,)), n_sem((NC,)),
            n_sem((NC,)), n_sem((NC,)), n_sem((NC,)), n_sem((NC,)),
            n_sem((NC // 2,)), n_sem((NC // 2,)),
            n_sem((NC // 2,)), n_sem((NC // 2,)),
        ],
        compiler_params=pltpu.CompilerParams(collective_id=0),
    )(p_hbm, g_hbm)
